# baseline (speedup 1.0000x reference)
import functools

import jax
import jax.numpy as jnp
from jax.experimental import pallas as pl

N = 10000
DEG = 32
D = 128
BLK = 400


def _gat_block(a1_ref, a2_ref, ft_ref, out_ref):
    a1 = a1_ref[...]
    a2 = a2_ref[...]
    ft = ft_ref[...]
    a = a1 + a2
    l = jnp.where(a > 0, a, 0.01 * a)
    m = jnp.max(l, axis=1, keepdims=True)
    e = jnp.exp(l - m)
    w = e / jnp.sum(e, axis=1, keepdims=True)
    out_ref[...] = jnp.sum(w[:, :, None] * ft, axis=1)


@jax.jit
def kernel(a1, a2, ft):
    a2r = a2.reshape(N, DEG)
    grid = (N // BLK,)
    return pl.pallas_call(
        _gat_block,
        grid=grid,
        in_specs=[
            pl.BlockSpec((BLK, 1), lambda i: (i, 0)),
            pl.BlockSpec((BLK, DEG), lambda i: (i, 0)),
            pl.BlockSpec((BLK, DEG, D), lambda i: (i, 0, 0)),
        ],
        out_specs=pl.BlockSpec((BLK, D), lambda i: (i, 0)),
        out_shape=jax.ShapeDtypeStruct((N, D), jnp.float32),
    )(a1, a2r, ft)
